# baseline (speedup 1.0000x reference)
import functools

import jax
import jax.numpy as jnp
from jax.experimental import pallas as pl

_NUM_LAYER = 3
_EMB = 128


def _bn_apply_kernel(y_ref, scale_ref, shift_ref, o_ref, *, do_relu):
    out = y_ref[...] * scale_ref[...] + shift_ref[...]
    if do_relu:
        out = jnp.maximum(out, 0.0)
    o_ref[...] = out


def _bn(h, g, b, do_relu):
    m = h.mean(axis=0)
    v = h.var(axis=0)
    scale = g / jnp.sqrt(v + 1e-5)
    shift = b - m * scale
    n = h.shape[0]
    bm = 2000
    grid = (n + bm - 1) // bm
    return pl.pallas_call(
        functools.partial(_bn_apply_kernel, do_relu=do_relu),
        grid=(grid,),
        in_specs=[
            pl.BlockSpec((bm, _EMB), lambda i: (i, 0)),
            pl.BlockSpec((1, _EMB), lambda i: (0, 0)),
            pl.BlockSpec((1, _EMB), lambda i: (0, 0)),
        ],
        out_specs=pl.BlockSpec((bm, _EMB), lambda i: (i, 0)),
        out_shape=jax.ShapeDtypeStruct((n, _EMB), jnp.float32),
    )(h, scale.reshape(1, _EMB), shift.reshape(1, _EMB))


def _gin_conv(h, ei, ea, p, self_idx):
    N = h.shape[0]
    ar = jnp.arange(N, dtype=ei.dtype)
    ei2 = jnp.concatenate([ei, jnp.stack([ar, ar])], axis=1)
    if ea.shape[1] != _EMB:
        ea = p['ee1'][ea[:, 0]] + p['ee2'][ea[:, 1]]
    sl = p['ee1'][self_idx] + p['ee2'][0]
    ea2 = jnp.concatenate([ea, jnp.broadcast_to(sl, (N, _EMB))], axis=0)
    msg = h[ei2[0]] + ea2
    agg = jax.ops.segment_sum(msg, ei2[1], num_segments=N)
    hid = jax.nn.relu(agg @ p['W1'] + p['b1'])
    return hid @ p['W2'] + p['b2']


def kernel(x, edge_index, edge_attr, lg_x, lg_edge_index, lg_edge_index_map, lg_edge_index_map2, params):
    h0 = params['node_emb1'][x[:, 0]] + params['node_emb2'][x[:, 1]]
    e0 = params['edge_emb1'][lg_x[:, 0]] + params['edge_emb2'][lg_x[:, 1]]
    lg_edge_attr = x[lg_edge_index_map2]
    h_list = [h0]
    e_list = [e0]
    ei = edge_index
    ea = edge_attr
    for layer in range(_NUM_LAYER):
        p = params['layers'][layer]
        h = _gin_conv(h_list[layer], ei, ea, p, 4)
        h = _bn(h, p['bn_g'], p['bn_b'], layer < _NUM_LAYER - 1)
        h_list.append(h)
        q = params['lg_layers'][layer]
        e = _gin_conv(e_list[layer], lg_edge_index, lg_edge_attr, q, 119)
        e = _bn(e, q['bn_g'], q['bn_b'], layer < _NUM_LAYER - 1)
        e_list.append(e)
        ei = jnp.concatenate([lg_edge_index_map, lg_edge_index_map[jnp.array([1, 0])]], axis=1)
        ea = jnp.concatenate([e, e], axis=0)
        lg_edge_attr = h[lg_edge_index_map2]
    return (tuple(h_list), tuple(e_list))


# speedup vs baseline: 1.2585x; 1.2585x over previous
import functools

import jax
import jax.numpy as jnp
from jax import lax
from jax.experimental import pallas as pl
from jax.experimental.pallas import tpu as pltpu
from jax.experimental.pallas import tpu_sc as plsc

NUM_LAYER = 3
EMB = 128
K = 64
CB = 128
NTILES = 16


@functools.cache
def _make_conv(E, Mpad, NCK, CH, W):
    assert E % (CB * NTILES) == 0 and NCK % 2 == 0 and CH % 128 == 0
    assert Mpad == NCK * CH and W % K == 0
    assert CH < (1 << 14) and (E // NTILES) <= (1 << 15)
    ET = E // NTILES
    HALF = NCK // 2
    RPT = CH // NTILES
    CAPT = 16 * W + 16
    SR = 16 * W
    R = ET // SR
    assert R * SR == ET
    NG = SR // CB

    mesh = plsc.VectorSubcoreMesh(core_axis_name="c", subcore_axis_name="s")

    @functools.partial(
        pl.kernel,
        out_type=jax.ShapeDtypeStruct((Mpad, EMB), jnp.float32),
        mesh=mesh,
        scratch_types=[
            pltpu.VMEM((ET,), jnp.int32),
            pltpu.VMEM((CB,), jnp.int32),
            pltpu.VMEM((CB,), jnp.int32),
            pltpu.VMEM((16 * W,), jnp.int32),
            pltpu.VMEM((K,), jnp.int32),
            pltpu.VMEM((K,), jnp.int32),
            pltpu.VMEM((K,), jnp.int32),
            pltpu.VMEM((K,), jnp.int32),
            pltpu.VMEM((K, EMB), jnp.float32),
            pltpu.VMEM((K, EMB), jnp.float32),
            pltpu.VMEM_SHARED((16 * CAPT,), jnp.int32),
            pltpu.VMEM_SHARED((CH + 8, EMB), jnp.float32),
            pltpu.SemaphoreType.DMA,
            pltpu.SemaphoreType.DMA,
        ],
    )
    def conv(a_hbm, b_hbm, s1_hbm, s2_hbm, dst_hbm, zrs_hbm, out_hbm,
             dstloc, posbuf, pkbuf, pkall,
             offblk, s1blk, s2blk, dstblk, rowsA, rowsB,
             cpk_sh, acc, semA, semB):
        cid = lax.axis_index("c")
        sid = lax.axis_index("s")
        ebase = sid * ET
        tbase = sid * CAPT
        pltpu.sync_copy(dst_hbm.at[pl.ds(ebase, ET)], dstloc)

        lanes = lax.iota(jnp.int32, 16)
        lane_base = lanes * W + tbase
        trash = jnp.full((16,), 16 * W, jnp.int32) + tbase
        dumppk = jnp.full((16,), CH << 15, jnp.int32)

        def unpack(src, off):
            for t in range(K // 16):
                pk = src[pl.ds(off + t * 16, 16)]
                rel = lax.shift_right_logical(pk, 15)
                dstblk[pl.ds(t * 16, 16)] = jnp.minimum(rel, CH)
                ofs = jnp.minimum(pk & 32767, ET - 1)
                offblk[pl.ds(t * 16, 16)] = ofs + ebase

        def do_block():
            ga = pltpu.async_copy(s1_hbm.at[offblk], s1blk, semA)
            gb = pltpu.async_copy(s2_hbm.at[offblk], s2blk, semB)
            ga.wait()
            gb.wait()
            ga = pltpu.async_copy(a_hbm.at[s1blk], rowsA, semA)
            gb = pltpu.async_copy(b_hbm.at[s2blk], rowsB, semB)
            ga.wait()
            gb.wait()
            sa = pltpu.async_copy(rowsA, acc.at[dstblk], semA, add=True)
            sa.wait()
            sb = pltpu.async_copy(rowsB, acc.at[dstblk], semB, add=True)
            sb.wait()

        def chunk_body(ci, carry):
            lo = (cid * HALF + ci) * CH
            pltpu.sync_copy(zrs_hbm.at[pl.ds(sid * RPT, RPT)],
                            acc.at[pl.ds(sid * RPT, RPT)])
            plsc.subcore_barrier()

            def round_body(r, rcarry):
                rbase = r * SR

                def cgroup(g, cnt_v):
                    i = rbase + g * CB
                    for u in range(CB // 16):
                        dv = dstloc[pl.ds(i + u * 16, 16)]
                        rel = dv - lo
                        t = rel | (CH - 1 - rel)
                        mi = lax.shift_right_arithmetic(t, 31) + 1
                        pos = trash + (lane_base + cnt_v - trash) * mi
                        posbuf[pl.ds(u * 16, 16)] = pos
                        pkbuf[pl.ds(u * 16, 16)] = (
                            lax.shift_left(rel, 15)
                            | (lanes + (i + u * 16)))
                        cnt_v = cnt_v + mi
                    pltpu.async_copy(pkbuf, cpk_sh.at[posbuf], semA).wait()
                    return cnt_v

                cnt_v = lax.fori_loop(0, NG, cgroup,
                                      jnp.zeros((16,), jnp.int32))

                def pgroup(pg, _):
                    for u in range(CB // 16):
                        pv = cnt_v + (pg * (CB // 16) + u)
                        t3 = (W - 1) - pv
                        mi2 = lax.shift_right_arithmetic(t3, 31) + 1
                        pos = trash + (lane_base + pv - trash) * mi2
                        posbuf[pl.ds(u * 16, 16)] = pos
                        pkbuf[pl.ds(u * 16, 16)] = dumppk
                    pltpu.async_copy(pkbuf, cpk_sh.at[posbuf], semA).wait()
                    return 0

                lax.fori_loop(0, K // (CB // 16), pgroup, 0)

                cps = [pltpu.async_copy(
                    cpk_sh.at[pl.ds(tbase + lane * W, W)],
                    pkall.at[pl.ds(lane * W, W)], semB)
                    for lane in range(16)]
                for cp in cps:
                    cp.wait()

                def block_b(b, lane_off):
                    unpack(pkall, lane_off + b * K)
                    do_block()
                    return lane_off

                for lane in range(16):
                    nblk = (cnt_v[lane] + (K - 1)) // K
                    lax.fori_loop(0, nblk, block_b, lane * W)
                return rcarry

            lax.fori_loop(0, R, round_body, 0)

            plsc.subcore_barrier()
            pltpu.sync_copy(acc.at[pl.ds(sid * RPT, RPT)],
                            out_hbm.at[pl.ds(lo + sid * RPT, RPT)])
            plsc.subcore_barrier()
            return carry

        lax.fori_loop(0, HALF, chunk_body, 0)

    return conv


def _embed_body(c1_ref, c2_ref, ta_ref, tb_ref, o_ref):
    i1 = lax.broadcasted_iota(jnp.int32, (c1_ref.shape[0], EMB), 1)
    oh1 = (c1_ref[...] == i1).astype(jnp.float32)
    oh2 = (c2_ref[...] == i1).astype(jnp.float32)
    o_ref[...] = (jnp.dot(oh1, ta_ref[...], preferred_element_type=jnp.float32, precision=lax.Precision.HIGHEST)
                  + jnp.dot(oh2, tb_ref[...], preferred_element_type=jnp.float32, precision=lax.Precision.HIGHEST))


def _embed(c1, c2, ta, tb, bm=2000):
    n = c1.shape[0]
    nb = n // bm
    assert nb * bm == n
    tap = jnp.zeros((EMB, EMB), jnp.float32).at[:ta.shape[0]].set(ta)
    tbp = jnp.zeros((EMB, EMB), jnp.float32).at[:tb.shape[0]].set(tb)
    return pl.pallas_call(
        _embed_body,
        grid=(nb,),
        in_specs=[
            pl.BlockSpec((bm, 1), lambda i: (i, 0)),
            pl.BlockSpec((bm, 1), lambda i: (i, 0)),
            pl.BlockSpec((EMB, EMB), lambda i: (0, 0)),
            pl.BlockSpec((EMB, EMB), lambda i: (0, 0)),
        ],
        out_specs=pl.BlockSpec((bm, EMB), lambda i: (i, 0)),
        out_shape=jax.ShapeDtypeStruct((n, EMB), jnp.float32),
    )(c1.reshape(n, 1), c2.reshape(n, 1), tap, tbp)


def _mlp_body(agg_ref, own_ref, sl_ref, w1_ref, b1_ref, w2_ref, b2_ref,
              y_ref, sums_ref):
    i = pl.program_id(0)
    pre = agg_ref[...] + own_ref[...] + sl_ref[...]
    hid = jnp.dot(pre, w1_ref[...], preferred_element_type=jnp.float32)
    hid = jnp.maximum(hid + b1_ref[...], 0.0)
    y = jnp.dot(hid, w2_ref[...], preferred_element_type=jnp.float32)
    y = y + b2_ref[...]
    y_ref[...] = y

    @pl.when(i == 0)
    def _():
        sums_ref[...] = jnp.zeros_like(sums_ref)

    sums_ref[0:1, :] += jnp.sum(y, axis=0, keepdims=True)
    sums_ref[1:2, :] += jnp.sum(y * y, axis=0, keepdims=True)


def _mlp(agg, own, sl, w1, b1, w2, b2, n_rows, bm=2000):
    nb = n_rows // bm
    assert nb * bm == n_rows
    y, sums = pl.pallas_call(
        _mlp_body,
        grid=(nb,),
        in_specs=[
            pl.BlockSpec((bm, EMB), lambda i: (i, 0)),
            pl.BlockSpec((bm, EMB), lambda i: (i, 0)),
            pl.BlockSpec((1, EMB), lambda i: (0, 0)),
            pl.BlockSpec((EMB, 2 * EMB), lambda i: (0, 0)),
            pl.BlockSpec((1, 2 * EMB), lambda i: (0, 0)),
            pl.BlockSpec((2 * EMB, EMB), lambda i: (0, 0)),
            pl.BlockSpec((1, EMB), lambda i: (0, 0)),
        ],
        out_specs=[
            pl.BlockSpec((bm, EMB), lambda i: (i, 0)),
            pl.BlockSpec((2, EMB), lambda i: (0, 0)),
        ],
        out_shape=[
            jax.ShapeDtypeStruct((n_rows, EMB), jnp.float32),
            jax.ShapeDtypeStruct((2, EMB), jnp.float32),
        ],
    )(agg, own, sl.reshape(1, EMB), w1, b1.reshape(1, 2 * EMB), w2,
      b2.reshape(1, EMB))
    return y, sums


def _css_body(y_ref, m_ref, o_ref):
    i = pl.program_id(0)
    d = y_ref[...] - m_ref[...]

    @pl.when(i == 0)
    def _():
        o_ref[...] = jnp.zeros_like(o_ref)

    o_ref[...] += jnp.sum(d * d, axis=0, keepdims=True)


def _colsumsq_centered(y, m, bm=2000):
    n = y.shape[0]
    nb = n // bm
    assert nb * bm == n
    return pl.pallas_call(
        _css_body,
        grid=(nb,),
        in_specs=[
            pl.BlockSpec((bm, EMB), lambda i: (i, 0)),
            pl.BlockSpec((1, EMB), lambda i: (0, 0)),
        ],
        out_specs=pl.BlockSpec((1, EMB), lambda i: (0, 0)),
        out_shape=jax.ShapeDtypeStruct((1, EMB), jnp.float32),
    )(y, m.reshape(1, EMB))


def _bn_apply_body(y_ref, sc_ref, sh_ref, o_ref, *, do_relu):
    out = y_ref[...] * sc_ref[...] + sh_ref[...]
    if do_relu:
        out = jnp.maximum(out, 0.0)
    o_ref[...] = out


def _bn_apply(y, scale, shift, do_relu, bm=2000):
    n = y.shape[0]
    nb = n // bm
    assert nb * bm == n
    return pl.pallas_call(
        functools.partial(_bn_apply_body, do_relu=do_relu),
        grid=(nb,),
        in_specs=[
            pl.BlockSpec((bm, EMB), lambda i: (i, 0)),
            pl.BlockSpec((1, EMB), lambda i: (0, 0)),
            pl.BlockSpec((1, EMB), lambda i: (0, 0)),
        ],
        out_specs=pl.BlockSpec((bm, EMB), lambda i: (i, 0)),
        out_shape=jax.ShapeDtypeStruct((n, EMB), jnp.float32),
    )(y, scale.reshape(1, EMB), shift.reshape(1, EMB))


def _bn_scale_shift(sums, y, n_rows, g, b):
    m = sums[0] / n_rows
    v = _colsumsq_centered(y, m)[0] / n_rows
    scale = g / jnp.sqrt(v + 1e-5)
    shift = b - m * scale
    return scale, shift


def _pad1(a, n, val):
    return jnp.pad(a, (0, n - a.shape[0]), constant_values=val)


def kernel(x, edge_index, edge_attr, lg_x, lg_edge_index, lg_edge_index_map,
           lg_edge_index_map2, params):
    i32 = jnp.int32
    x = x.astype(i32)
    edge_index = edge_index.astype(i32)
    edge_attr = edge_attr.astype(i32)
    lg_x = lg_x.astype(i32)
    lg_edge_index = lg_edge_index.astype(i32)
    lg_edge_index_map = lg_edge_index_map.astype(i32)
    lg_edge_index_map2 = lg_edge_index_map2.astype(i32)

    N = x.shape[0]
    M = lg_x.shape[0]

    NP, N_NCK, N_CH, N_W = 10240, 2, 5120, 320
    MP, M_NCK, M_CH, M_W = 80896, 8, 10112, 320
    EP = 163840
    ELGP = 327680
    conv_node = _make_conv(EP, NP, N_NCK, N_CH, N_W)
    conv_lg = _make_conv(ELGP, MP, M_NCK, M_CH, M_W)

    zrs = jnp.zeros((max(N_CH, M_CH), EMB), jnp.float32)
    arM = jnp.arange(M, dtype=i32)

    h0 = _embed(x[:, 0], x[:, 1], params['node_emb1'], params['node_emb2'])
    e0 = _embed(lg_x[:, 0], lg_x[:, 1], params['edge_emb1'],
                params['edge_emb2'])

    code_n0 = _pad1(edge_attr[:, 0] * 3 + edge_attr[:, 1], EP, 0)
    xg = x[lg_edge_index_map2]
    code_lg0 = _pad1(xg[:, 0] * 3 + xg[:, 1], ELGP, 0)
    ns1_0 = _pad1(edge_index[0], EP, 0)
    ndst_0 = _pad1(edge_index[1], EP, -1)
    ns1_12 = _pad1(jnp.concatenate([lg_edge_index_map[0], lg_edge_index_map[1]]), EP, 0)
    ndst_12 = _pad1(jnp.concatenate([lg_edge_index_map[1], lg_edge_index_map[0]]), EP, -1)
    ns2_12 = _pad1(jnp.concatenate([arM, arM]), EP, 0)
    lgs1 = _pad1(lg_edge_index[0], ELGP, 0)
    lgdst = _pad1(lg_edge_index[1], ELGP, -1)
    lgmap2 = _pad1(lg_edge_index_map2, ELGP, 0)

    h_cur = h0
    e_cur = e0
    h_list = [h0]
    e_list = [e0]

    for layer in range(NUM_LAYER):
        p = params['layers'][layer]
        q = params['lg_layers'][layer]
        do_relu = layer < NUM_LAYER - 1

        if layer == 0:
            t_n = (p['ee1'][:6, None, :] + p['ee2'][None, :3, :]).reshape(18, EMB)
            aggh = conv_node(h_cur, t_n, ns1_0, code_n0, ndst_0, zrs)
        else:
            aggh = conv_node(h_cur, e_cur, ns1_12, ns2_12, ndst_12, zrs)
        sl_n = p['ee1'][4] + p['ee2'][0]
        y_h, sums_h = _mlp(aggh, h_cur, sl_n, p['W1'], p['b1'], p['W2'],
                           p['b2'], N)
        sc_h, sh_h = _bn_scale_shift(sums_h, y_h, N, p['bn_g'], p['bn_b'])
        h_new = _bn_apply(y_h, sc_h, sh_h, do_relu)

        if layer == 0:
            t_l = (q['ee1'][:3, None, :] + q['ee2'][None, :3, :]).reshape(9, EMB)
            agge = conv_lg(e_cur, t_l, lgs1, code_lg0, lgdst, zrs)
        else:
            agge = conv_lg(e_cur, h_cur, lgs1, lgmap2, lgdst, zrs)
        sl_l = q['ee1'][119] + q['ee2'][0]
        y_e, sums_e = _mlp(agge, e_cur, sl_l, q['W1'], q['b1'], q['W2'],
                           q['b2'], M)
        sc_e, sh_e = _bn_scale_shift(sums_e, y_e, M, q['bn_g'], q['bn_b'])
        e_new = _bn_apply(y_e, sc_e, sh_e, do_relu)

        h_cur = h_new
        e_cur = e_new
        h_list.append(h_new)
        e_list.append(e_new)

    return (tuple(h_list), tuple(e_list))


# speedup vs baseline: 1.2717x; 1.0105x over previous
import functools

import jax
import jax.numpy as jnp
from jax import lax
from jax.experimental import pallas as pl
from jax.experimental.pallas import tpu as pltpu
from jax.experimental.pallas import tpu_sc as plsc

NUM_LAYER = 3
EMB = 128
K = 64
CB = 128
NTILES = 16


@functools.cache
def _make_conv(E, Mpad, NCK, CH, W):
    assert E % (CB * NTILES) == 0 and NCK % 2 == 0 and CH % 128 == 0
    assert Mpad == NCK * CH and W % K == 0
    assert CH < (1 << 14) and (E // NTILES) <= (1 << 15)
    ET = E // NTILES
    HALF = NCK // 2
    RPT = CH // NTILES
    CAPT = 16 * W + 16
    SR = 16 * W
    R = ET // SR
    assert R * SR == ET
    NG = SR // CB

    mesh = plsc.VectorSubcoreMesh(core_axis_name="c", subcore_axis_name="s")

    @functools.partial(
        pl.kernel,
        out_type=jax.ShapeDtypeStruct((Mpad, EMB), jnp.float32),
        mesh=mesh,
        scratch_types=[
            pltpu.VMEM((1024,), jnp.int32),
            pltpu.VMEM((CB,), jnp.int32),
            pltpu.VMEM((CB,), jnp.int32),
            pltpu.VMEM((16 * W,), jnp.int32),
            pltpu.VMEM((K,), jnp.int32),
            pltpu.VMEM((K,), jnp.int32),
            pltpu.VMEM((K,), jnp.int32),
            pltpu.VMEM((K,), jnp.int32),
            pltpu.VMEM((K, EMB), jnp.float32),
            pltpu.VMEM((K, EMB), jnp.float32),
            pltpu.VMEM((K,), jnp.int32),
            pltpu.VMEM((K,), jnp.int32),
            pltpu.VMEM((K,), jnp.int32),
            pltpu.VMEM((K,), jnp.int32),
            pltpu.VMEM((K, EMB), jnp.float32),
            pltpu.VMEM((K, EMB), jnp.float32),
            pltpu.VMEM_SHARED((16 * CAPT,), jnp.int32),
            pltpu.VMEM_SHARED((CH + 8, EMB), jnp.float32),
            pltpu.SemaphoreType.DMA,
            pltpu.SemaphoreType.DMA,
            pltpu.SemaphoreType.DMA,
            pltpu.SemaphoreType.DMA,
            pltpu.SemaphoreType.DMA,
            pltpu.SemaphoreType.DMA,
        ],
    )
    def conv(a_hbm, b_hbm, s1_hbm, s2_hbm, dst_hbm, zrs_hbm, out_hbm,
             dstwin, posbuf, pkbuf, pkall,
             offblk0, s1blk0, s2blk0, dstblk0, rowsA0, rowsB0,
             offblk1, s1blk1, s2blk1, dstblk1, rowsA1, rowsB1,
             cpk_sh, acc, semA, semB, semC, semD, semE, semF):
        cid = lax.axis_index("c")
        sid = lax.axis_index("s")
        ebase = sid * ET
        tbase = sid * CAPT

        offblk = [offblk0, offblk1]
        s1blk = [s1blk0, s1blk1]
        s2blk = [s2blk0, s2blk1]
        dstblk = [dstblk0, dstblk1]
        rowsA = [rowsA0, rowsA1]
        rowsB = [rowsB0, rowsB1]

        lanes = lax.iota(jnp.int32, 16)
        lane_base = lanes * W + tbase
        trash = jnp.full((16,), 16 * W, jnp.int32) + tbase
        dumppk = jnp.full((16,), CH << 15, jnp.int32)

        def unpack(src, off, p):
            for t in range(K // 16):
                pk = src[pl.ds(off + t * 16, 16)]
                rel = lax.shift_right_logical(pk, 15)
                dstblk[p][pl.ds(t * 16, 16)] = jnp.minimum(rel, CH)
                ofs = jnp.minimum(pk & 32767, ET - 1)
                offblk[p][pl.ds(t * 16, 16)] = ofs + ebase

        def fire_ints(p):
            return (pltpu.async_copy(s1_hbm.at[offblk[p]], s1blk[p], semA),
                    pltpu.async_copy(s2_hbm.at[offblk[p]], s2blk[p], semB))

        def fire_rows(p, ints):
            ints[0].wait()
            ints[1].wait()
            return (pltpu.async_copy(a_hbm.at[s1blk[p]], rowsA[p], semC),
                    pltpu.async_copy(b_hbm.at[s2blk[p]], rowsB[p], semD))

        def scatters(p, rows):
            rows[0].wait()
            rows[1].wait()
            sa = pltpu.async_copy(rowsA[p], acc.at[dstblk[p]], semE, add=True)
            sa.wait()
            sb = pltpu.async_copy(rowsB[p], acc.at[dstblk[p]], semF, add=True)
            sb.wait()

        def do_block(p):
            ints = fire_ints(p)
            rows = fire_rows(p, ints)
            scatters(p, rows)

        def chunk_body(ci, carry):
            lo = (cid * HALF + ci) * CH
            pltpu.sync_copy(zrs_hbm.at[pl.ds(sid * RPT, RPT)],
                            acc.at[pl.ds(sid * RPT, RPT)])
            plsc.subcore_barrier()

            def round_body(r, rcarry):
                rbase = r * SR

                cnt_v = jnp.zeros((16,), jnp.int32)
                for w in range(SR // 1024):
                    wbase = rbase + w * 1024
                    pltpu.sync_copy(dst_hbm.at[pl.ds(ebase + wbase, 1024)],
                                    dstwin)

                    def cgroup(g, cnt_v):
                        i = g * CB
                        for u in range(CB // 16):
                            dv = dstwin[pl.ds(i + u * 16, 16)]
                            rel = dv - lo
                            t = rel | (CH - 1 - rel)
                            mi = lax.shift_right_arithmetic(t, 31) + 1
                            pos = trash + (lane_base + cnt_v - trash) * mi
                            posbuf[pl.ds(u * 16, 16)] = pos
                            pkbuf[pl.ds(u * 16, 16)] = (
                                lax.shift_left(rel, 15)
                                | (lanes + (wbase + i + u * 16)))
                            cnt_v = cnt_v + mi
                        pltpu.async_copy(pkbuf, cpk_sh.at[posbuf],
                                         semA).wait()
                        return cnt_v

                    cnt_v = lax.fori_loop(0, 1024 // CB, cgroup, cnt_v)

                def pgroup(pg, _):
                    for u in range(CB // 16):
                        pv = cnt_v + (pg * (CB // 16) + u)
                        t3 = (W - 1) - pv
                        mi2 = lax.shift_right_arithmetic(t3, 31) + 1
                        pos = trash + (lane_base + pv - trash) * mi2
                        posbuf[pl.ds(u * 16, 16)] = pos
                        pkbuf[pl.ds(u * 16, 16)] = dumppk
                    pltpu.async_copy(pkbuf, cpk_sh.at[posbuf], semA).wait()
                    return 0

                lax.fori_loop(0, K // (CB // 16), pgroup, 0)

                cps = [pltpu.async_copy(
                    cpk_sh.at[pl.ds(tbase + lane * W, W)],
                    pkall.at[pl.ds(lane * W, W)], semB)
                    for lane in range(16)]
                for cp in cps:
                    cp.wait()

                ints = {}
                rows = {}
                for l in range(18):
                    if 2 <= l:
                        scatters((l - 2) & 1, rows[l - 2])
                    if 1 <= l <= 16:
                        rows[l - 1] = fire_rows((l - 1) & 1, ints[l - 1])
                    if l < 16:
                        unpack(pkall, l * W, l & 1)
                        ints[l] = fire_ints(l & 1)

                def block_b(b, lane_off):
                    unpack(pkall, lane_off + b * K, 0)
                    do_block(0)
                    return lane_off

                for lane in range(16):
                    nblk = (cnt_v[lane] + (K - 1)) // K
                    lax.fori_loop(1, jnp.maximum(nblk, 1), block_b, lane * W)
                return rcarry

            lax.fori_loop(0, R, round_body, 0)

            plsc.subcore_barrier()
            pltpu.sync_copy(acc.at[pl.ds(sid * RPT, RPT)],
                            out_hbm.at[pl.ds(lo + sid * RPT, RPT)])
            plsc.subcore_barrier()
            return carry

        lax.fori_loop(0, HALF, chunk_body, 0)

    return conv


def _embed_body(c1_ref, c2_ref, ta_ref, tb_ref, o_ref):
    i1 = lax.broadcasted_iota(jnp.int32, (c1_ref.shape[0], EMB), 1)
    oh1 = (c1_ref[...] == i1).astype(jnp.float32)
    oh2 = (c2_ref[...] == i1).astype(jnp.float32)
    o_ref[...] = (jnp.dot(oh1, ta_ref[...], preferred_element_type=jnp.float32, precision=lax.Precision.HIGHEST)
                  + jnp.dot(oh2, tb_ref[...], preferred_element_type=jnp.float32, precision=lax.Precision.HIGHEST))


def _embed(c1, c2, ta, tb, bm=2000):
    n = c1.shape[0]
    nb = n // bm
    assert nb * bm == n
    tap = jnp.zeros((EMB, EMB), jnp.float32).at[:ta.shape[0]].set(ta)
    tbp = jnp.zeros((EMB, EMB), jnp.float32).at[:tb.shape[0]].set(tb)
    return pl.pallas_call(
        _embed_body,
        grid=(nb,),
        in_specs=[
            pl.BlockSpec((bm, 1), lambda i: (i, 0)),
            pl.BlockSpec((bm, 1), lambda i: (i, 0)),
            pl.BlockSpec((EMB, EMB), lambda i: (0, 0)),
            pl.BlockSpec((EMB, EMB), lambda i: (0, 0)),
        ],
        out_specs=pl.BlockSpec((bm, EMB), lambda i: (i, 0)),
        out_shape=jax.ShapeDtypeStruct((n, EMB), jnp.float32),
    )(c1.reshape(n, 1), c2.reshape(n, 1), tap, tbp)


def _mlp_body(agg_ref, own_ref, sl_ref, w1_ref, b1_ref, w2_ref, b2_ref,
              y_ref, sums_ref):
    i = pl.program_id(0)
    pre = agg_ref[...] + own_ref[...] + sl_ref[...]
    hid = jnp.dot(pre, w1_ref[...], preferred_element_type=jnp.float32)
    hid = jnp.maximum(hid + b1_ref[...], 0.0)
    y = jnp.dot(hid, w2_ref[...], preferred_element_type=jnp.float32)
    y = y + b2_ref[...]
    y_ref[...] = y

    @pl.when(i == 0)
    def _():
        sums_ref[...] = jnp.zeros_like(sums_ref)

    sums_ref[0:1, :] += jnp.sum(y, axis=0, keepdims=True)
    sums_ref[1:2, :] += jnp.sum(y * y, axis=0, keepdims=True)


def _mlp(agg, own, sl, w1, b1, w2, b2, n_rows, bm=2000):
    nb = n_rows // bm
    assert nb * bm == n_rows
    y, sums = pl.pallas_call(
        _mlp_body,
        grid=(nb,),
        in_specs=[
            pl.BlockSpec((bm, EMB), lambda i: (i, 0)),
            pl.BlockSpec((bm, EMB), lambda i: (i, 0)),
            pl.BlockSpec((1, EMB), lambda i: (0, 0)),
            pl.BlockSpec((EMB, 2 * EMB), lambda i: (0, 0)),
            pl.BlockSpec((1, 2 * EMB), lambda i: (0, 0)),
            pl.BlockSpec((2 * EMB, EMB), lambda i: (0, 0)),
            pl.BlockSpec((1, EMB), lambda i: (0, 0)),
        ],
        out_specs=[
            pl.BlockSpec((bm, EMB), lambda i: (i, 0)),
            pl.BlockSpec((2, EMB), lambda i: (0, 0)),
        ],
        out_shape=[
            jax.ShapeDtypeStruct((n_rows, EMB), jnp.float32),
            jax.ShapeDtypeStruct((2, EMB), jnp.float32),
        ],
    )(agg, own, sl.reshape(1, EMB), w1, b1.reshape(1, 2 * EMB), w2,
      b2.reshape(1, EMB))
    return y, sums


def _css_body(y_ref, m_ref, o_ref):
    i = pl.program_id(0)
    d = y_ref[...] - m_ref[...]

    @pl.when(i == 0)
    def _():
        o_ref[...] = jnp.zeros_like(o_ref)

    o_ref[...] += jnp.sum(d * d, axis=0, keepdims=True)


def _colsumsq_centered(y, m, bm=2000):
    n = y.shape[0]
    nb = n // bm
    assert nb * bm == n
    return pl.pallas_call(
        _css_body,
        grid=(nb,),
        in_specs=[
            pl.BlockSpec((bm, EMB), lambda i: (i, 0)),
            pl.BlockSpec((1, EMB), lambda i: (0, 0)),
        ],
        out_specs=pl.BlockSpec((1, EMB), lambda i: (0, 0)),
        out_shape=jax.ShapeDtypeStruct((1, EMB), jnp.float32),
    )(y, m.reshape(1, EMB))


def _bn_apply_body(y_ref, sc_ref, sh_ref, o_ref, *, do_relu):
    out = y_ref[...] * sc_ref[...] + sh_ref[...]
    if do_relu:
        out = jnp.maximum(out, 0.0)
    o_ref[...] = out


def _bn_apply(y, scale, shift, do_relu, bm=2000):
    n = y.shape[0]
    nb = n // bm
    assert nb * bm == n
    return pl.pallas_call(
        functools.partial(_bn_apply_body, do_relu=do_relu),
        grid=(nb,),
        in_specs=[
            pl.BlockSpec((bm, EMB), lambda i: (i, 0)),
            pl.BlockSpec((1, EMB), lambda i: (0, 0)),
            pl.BlockSpec((1, EMB), lambda i: (0, 0)),
        ],
        out_specs=pl.BlockSpec((bm, EMB), lambda i: (i, 0)),
        out_shape=jax.ShapeDtypeStruct((n, EMB), jnp.float32),
    )(y, scale.reshape(1, EMB), shift.reshape(1, EMB))


def _bn_scale_shift(sums, y, n_rows, g, b):
    m = sums[0] / n_rows
    v = _colsumsq_centered(y, m)[0] / n_rows
    scale = g / jnp.sqrt(v + 1e-5)
    shift = b - m * scale
    return scale, shift


def _pad1(a, n, val):
    return jnp.pad(a, (0, n - a.shape[0]), constant_values=val)


def kernel(x, edge_index, edge_attr, lg_x, lg_edge_index, lg_edge_index_map,
           lg_edge_index_map2, params):
    i32 = jnp.int32
    x = x.astype(i32)
    edge_index = edge_index.astype(i32)
    edge_attr = edge_attr.astype(i32)
    lg_x = lg_x.astype(i32)
    lg_edge_index = lg_edge_index.astype(i32)
    lg_edge_index_map = lg_edge_index_map.astype(i32)
    lg_edge_index_map2 = lg_edge_index_map2.astype(i32)

    N = x.shape[0]
    M = lg_x.shape[0]

    NP, N_NCK, N_CH, N_W = 10240, 2, 5120, 320
    MP, M_NCK, M_CH, M_W = 80896, 8, 10112, 320
    EP = 163840
    ELGP = 327680
    conv_node = _make_conv(EP, NP, N_NCK, N_CH, N_W)
    conv_lg = _make_conv(ELGP, MP, M_NCK, M_CH, M_W)

    zrs = jnp.zeros((max(N_CH, M_CH), EMB), jnp.float32)
    arM = jnp.arange(M, dtype=i32)

    h0 = _embed(x[:, 0], x[:, 1], params['node_emb1'], params['node_emb2'])
    e0 = _embed(lg_x[:, 0], lg_x[:, 1], params['edge_emb1'],
                params['edge_emb2'])

    code_n0 = _pad1(edge_attr[:, 0] * 3 + edge_attr[:, 1], EP, 0)
    xg = x[lg_edge_index_map2]
    code_lg0 = _pad1(xg[:, 0] * 3 + xg[:, 1], ELGP, 0)
    ns1_0 = _pad1(edge_index[0], EP, 0)
    ndst_0 = _pad1(edge_index[1], EP, -1)
    ns1_12 = _pad1(jnp.concatenate([lg_edge_index_map[0], lg_edge_index_map[1]]), EP, 0)
    ndst_12 = _pad1(jnp.concatenate([lg_edge_index_map[1], lg_edge_index_map[0]]), EP, -1)
    ns2_12 = _pad1(jnp.concatenate([arM, arM]), EP, 0)
    lgs1 = _pad1(lg_edge_index[0], ELGP, 0)
    lgdst = _pad1(lg_edge_index[1], ELGP, -1)
    lgmap2 = _pad1(lg_edge_index_map2, ELGP, 0)

    h_cur = h0
    e_cur = e0
    h_list = [h0]
    e_list = [e0]

    for layer in range(NUM_LAYER):
        p = params['layers'][layer]
        q = params['lg_layers'][layer]
        do_relu = layer < NUM_LAYER - 1

        if layer == 0:
            t_n = (p['ee1'][:6, None, :] + p['ee2'][None, :3, :]).reshape(18, EMB)
            aggh = conv_node(h_cur, t_n, ns1_0, code_n0, ndst_0, zrs)
        else:
            aggh = conv_node(h_cur, e_cur, ns1_12, ns2_12, ndst_12, zrs)
        sl_n = p['ee1'][4] + p['ee2'][0]
        y_h, sums_h = _mlp(aggh, h_cur, sl_n, p['W1'], p['b1'], p['W2'],
                           p['b2'], N)
        sc_h, sh_h = _bn_scale_shift(sums_h, y_h, N, p['bn_g'], p['bn_b'])
        h_new = _bn_apply(y_h, sc_h, sh_h, do_relu)

        if layer == 0:
            t_l = (q['ee1'][:3, None, :] + q['ee2'][None, :3, :]).reshape(9, EMB)
            agge = conv_lg(e_cur, t_l, lgs1, code_lg0, lgdst, zrs)
        else:
            agge = conv_lg(e_cur, h_cur, lgs1, lgmap2, lgdst, zrs)
        sl_l = q['ee1'][119] + q['ee2'][0]
        y_e, sums_e = _mlp(agge, e_cur, sl_l, q['W1'], q['b1'], q['W2'],
                           q['b2'], M)
        sc_e, sh_e = _bn_scale_shift(sums_e, y_e, M, q['bn_g'], q['bn_b'])
        e_new = _bn_apply(y_e, sc_e, sh_e, do_relu)

        h_cur = h_new
        e_cur = e_new
        h_list.append(h_new)
        e_list.append(e_new)

    return (tuple(h_list), tuple(e_list))


# speedup vs baseline: 1.2808x; 1.0072x over previous
import functools

import jax
import jax.numpy as jnp
from jax import lax
from jax.experimental import pallas as pl
from jax.experimental.pallas import tpu as pltpu
from jax.experimental.pallas import tpu_sc as plsc

NUM_LAYER = 3
EMB = 128
K = 64
CB = 128
NTILES = 16


@functools.cache
def _make_conv(E, Mpad, NCK, CH, W):
    assert E % (CB * NTILES) == 0 and NCK % 2 == 0 and CH % 128 == 0
    assert Mpad == NCK * CH and W % K == 0
    assert CH < (1 << 14) and (E // NTILES) <= (1 << 15)
    ET = E // NTILES
    HALF = NCK // 2
    RPT = CH // NTILES
    CAPT = 16 * W + 16
    SR = 16 * W
    R = ET // SR
    assert R * SR == ET
    NG = SR // CB

    mesh = plsc.VectorSubcoreMesh(core_axis_name="c", subcore_axis_name="s")

    @functools.partial(
        pl.kernel,
        out_type=jax.ShapeDtypeStruct((Mpad, EMB), jnp.float32),
        mesh=mesh,
        scratch_types=[
            pltpu.VMEM((1024,), jnp.int32),
            pltpu.VMEM((CB,), jnp.int32),
            pltpu.VMEM((CB,), jnp.int32),
            pltpu.VMEM((CB,), jnp.int32),
            pltpu.VMEM((CB,), jnp.int32),
            pltpu.VMEM((16 * W,), jnp.int32),
            pltpu.VMEM((K,), jnp.int32),
            pltpu.VMEM((K,), jnp.int32),
            pltpu.VMEM((K,), jnp.int32),
            pltpu.VMEM((K,), jnp.int32),
            pltpu.VMEM((K, EMB), jnp.float32),
            pltpu.VMEM((K, EMB), jnp.float32),
            pltpu.VMEM((K,), jnp.int32),
            pltpu.VMEM((K,), jnp.int32),
            pltpu.VMEM((K,), jnp.int32),
            pltpu.VMEM((K,), jnp.int32),
            pltpu.VMEM((K, EMB), jnp.float32),
            pltpu.VMEM((K, EMB), jnp.float32),
            pltpu.VMEM_SHARED((16 * CAPT,), jnp.int32),
            pltpu.VMEM_SHARED((CH + 8, EMB), jnp.float32),
            pltpu.SemaphoreType.DMA,
            pltpu.SemaphoreType.DMA,
            pltpu.SemaphoreType.DMA,
            pltpu.SemaphoreType.DMA,
            pltpu.SemaphoreType.DMA,
            pltpu.SemaphoreType.DMA,
        ],
    )
    def conv(a_hbm, b_hbm, s1_hbm, s2_hbm, dst_hbm, zrs_hbm, out_hbm,
             dstwin, posbuf0, pkbuf0, posbuf1, pkbuf1, pkall,
             offblk0, s1blk0, s2blk0, dstblk0, rowsA0, rowsB0,
             offblk1, s1blk1, s2blk1, dstblk1, rowsA1, rowsB1,
             cpk_sh, acc, semA, semB, semC, semD, semE, semF):
        cid = lax.axis_index("c")
        sid = lax.axis_index("s")
        ebase = sid * ET
        tbase = sid * CAPT

        posbuf = [posbuf0, posbuf1]
        pkbuf = [pkbuf0, pkbuf1]
        offblk = [offblk0, offblk1]
        s1blk = [s1blk0, s1blk1]
        s2blk = [s2blk0, s2blk1]
        dstblk = [dstblk0, dstblk1]
        rowsA = [rowsA0, rowsA1]
        rowsB = [rowsB0, rowsB1]

        lanes = lax.iota(jnp.int32, 16)
        lane_base = lanes * W + tbase
        trash = jnp.full((16,), 16 * W, jnp.int32) + tbase
        dumppk = jnp.full((16,), CH << 15, jnp.int32)

        def unpack(src, off, p):
            for t in range(K // 16):
                pk = src[pl.ds(off + t * 16, 16)]
                rel = lax.shift_right_logical(pk, 15)
                dstblk[p][pl.ds(t * 16, 16)] = jnp.minimum(rel, CH)
                ofs = jnp.minimum(pk & 32767, ET - 1)
                offblk[p][pl.ds(t * 16, 16)] = ofs + ebase

        def fire_ints(p):
            return (pltpu.async_copy(s1_hbm.at[offblk[p]], s1blk[p], semA),
                    pltpu.async_copy(s2_hbm.at[offblk[p]], s2blk[p], semB))

        def fire_rows(p, ints):
            ints[0].wait()
            ints[1].wait()
            return (pltpu.async_copy(a_hbm.at[s1blk[p]], rowsA[p], semC),
                    pltpu.async_copy(b_hbm.at[s2blk[p]], rowsB[p], semD))

        def scatters(p, rows):
            rows[0].wait()
            rows[1].wait()
            sa = pltpu.async_copy(rowsA[p], acc.at[dstblk[p]], semE, add=True)
            sa.wait()
            sb = pltpu.async_copy(rowsB[p], acc.at[dstblk[p]], semF, add=True)
            sb.wait()

        def do_block(p):
            ints = fire_ints(p)
            rows = fire_rows(p, ints)
            scatters(p, rows)

        def chunk_body(ci, carry):
            lo = (cid * HALF + ci) * CH
            pltpu.sync_copy(zrs_hbm.at[pl.ds(sid * RPT, RPT)],
                            acc.at[pl.ds(sid * RPT, RPT)])
            plsc.subcore_barrier()

            def round_body(r, rcarry):
                rbase = r * SR

                _drain = pltpu.make_async_copy(
                    s1_hbm.at[pl.ds(0, CB)], pkall.at[pl.ds(0, CB)], semA)
                pltpu.async_copy(s1_hbm.at[pl.ds(0, CB)],
                                 pkall.at[pl.ds(0, CB)], semA)
                pltpu.async_copy(s1_hbm.at[pl.ds(0, CB)],
                                 pkall.at[pl.ds(0, CB)], semA)

                cnt_v = jnp.zeros((16,), jnp.int32)
                for w in range(SR // 1024):
                    wbase = rbase + w * 1024
                    pltpu.sync_copy(dst_hbm.at[pl.ds(ebase + wbase, 1024)],
                                    dstwin)

                    for g in range(1024 // CB):
                        _drain.wait()
                        par = g & 1
                        i = g * CB
                        for u in range(CB // 16):
                            dv = dstwin[pl.ds(i + u * 16, 16)]
                            rel = dv - lo
                            t = rel | (CH - 1 - rel)
                            mi = lax.shift_right_arithmetic(t, 31) + 1
                            pos = trash + (lane_base + cnt_v - trash) * mi
                            posbuf[par][pl.ds(u * 16, 16)] = pos
                            pkbuf[par][pl.ds(u * 16, 16)] = (
                                lax.shift_left(rel, 15)
                                | (lanes + (wbase + i + u * 16)))
                            cnt_v = cnt_v + mi
                        pltpu.async_copy(pkbuf[par],
                                         cpk_sh.at[posbuf[par]], semA)
                _drain.wait()
                _drain.wait()

                def pgroup(pg, _):
                    for u in range(CB // 16):
                        pv = cnt_v + (pg * (CB // 16) + u)
                        t3 = (W - 1) - pv
                        mi2 = lax.shift_right_arithmetic(t3, 31) + 1
                        pos = trash + (lane_base + pv - trash) * mi2
                        posbuf[0][pl.ds(u * 16, 16)] = pos
                        pkbuf[0][pl.ds(u * 16, 16)] = dumppk
                    pltpu.async_copy(pkbuf[0], cpk_sh.at[posbuf[0]],
                                     semA).wait()
                    return 0

                lax.fori_loop(0, K // (CB // 16), pgroup, 0)

                cps = [pltpu.async_copy(
                    cpk_sh.at[pl.ds(tbase + lane * W, W)],
                    pkall.at[pl.ds(lane * W, W)], semB)
                    for lane in range(16)]
                for cp in cps:
                    cp.wait()

                ints = {}
                rows = {}
                for l in range(18):
                    if 2 <= l:
                        scatters((l - 2) & 1, rows[l - 2])
                    if 1 <= l <= 16:
                        rows[l - 1] = fire_rows((l - 1) & 1, ints[l - 1])
                    if l < 16:
                        unpack(pkall, l * W, l & 1)
                        ints[l] = fire_ints(l & 1)

                def block_b(b, lane_off):
                    unpack(pkall, lane_off + b * K, 0)
                    do_block(0)
                    return lane_off

                for lane in range(16):
                    nblk = (cnt_v[lane] + (K - 1)) // K
                    lax.fori_loop(1, jnp.maximum(nblk, 1), block_b, lane * W)
                return rcarry

            lax.fori_loop(0, R, round_body, 0)

            plsc.subcore_barrier()
            pltpu.sync_copy(acc.at[pl.ds(sid * RPT, RPT)],
                            out_hbm.at[pl.ds(lo + sid * RPT, RPT)])
            plsc.subcore_barrier()
            return carry

        lax.fori_loop(0, HALF, chunk_body, 0)

    return conv


def _embed_body(c1_ref, c2_ref, ta_ref, tb_ref, o_ref):
    i1 = lax.broadcasted_iota(jnp.int32, (c1_ref.shape[0], EMB), 1)
    oh1 = (c1_ref[...] == i1).astype(jnp.float32)
    oh2 = (c2_ref[...] == i1).astype(jnp.float32)
    o_ref[...] = (jnp.dot(oh1, ta_ref[...], preferred_element_type=jnp.float32, precision=lax.Precision.HIGHEST)
                  + jnp.dot(oh2, tb_ref[...], preferred_element_type=jnp.float32, precision=lax.Precision.HIGHEST))


def _embed(c1, c2, ta, tb, bm=2000):
    n = c1.shape[0]
    nb = n // bm
    assert nb * bm == n
    tap = jnp.zeros((EMB, EMB), jnp.float32).at[:ta.shape[0]].set(ta)
    tbp = jnp.zeros((EMB, EMB), jnp.float32).at[:tb.shape[0]].set(tb)
    return pl.pallas_call(
        _embed_body,
        grid=(nb,),
        in_specs=[
            pl.BlockSpec((bm, 1), lambda i: (i, 0)),
            pl.BlockSpec((bm, 1), lambda i: (i, 0)),
            pl.BlockSpec((EMB, EMB), lambda i: (0, 0)),
            pl.BlockSpec((EMB, EMB), lambda i: (0, 0)),
        ],
        out_specs=pl.BlockSpec((bm, EMB), lambda i: (i, 0)),
        out_shape=jax.ShapeDtypeStruct((n, EMB), jnp.float32),
    )(c1.reshape(n, 1), c2.reshape(n, 1), tap, tbp)


def _mlp_body(agg_ref, own_ref, sl_ref, w1_ref, b1_ref, w2_ref, b2_ref,
              y_ref, sums_ref):
    i = pl.program_id(0)
    pre = agg_ref[...] + own_ref[...] + sl_ref[...]
    hid = jnp.dot(pre, w1_ref[...], preferred_element_type=jnp.float32)
    hid = jnp.maximum(hid + b1_ref[...], 0.0)
    y = jnp.dot(hid, w2_ref[...], preferred_element_type=jnp.float32)
    y = y + b2_ref[...]
    y_ref[...] = y

    @pl.when(i == 0)
    def _():
        sums_ref[...] = jnp.zeros_like(sums_ref)

    sums_ref[0:1, :] += jnp.sum(y, axis=0, keepdims=True)
    sums_ref[1:2, :] += jnp.sum(y * y, axis=0, keepdims=True)


def _mlp(agg, own, sl, w1, b1, w2, b2, n_rows, bm=2000):
    nb = n_rows // bm
    assert nb * bm == n_rows
    y, sums = pl.pallas_call(
        _mlp_body,
        grid=(nb,),
        in_specs=[
            pl.BlockSpec((bm, EMB), lambda i: (i, 0)),
            pl.BlockSpec((bm, EMB), lambda i: (i, 0)),
            pl.BlockSpec((1, EMB), lambda i: (0, 0)),
            pl.BlockSpec((EMB, 2 * EMB), lambda i: (0, 0)),
            pl.BlockSpec((1, 2 * EMB), lambda i: (0, 0)),
            pl.BlockSpec((2 * EMB, EMB), lambda i: (0, 0)),
            pl.BlockSpec((1, EMB), lambda i: (0, 0)),
        ],
        out_specs=[
            pl.BlockSpec((bm, EMB), lambda i: (i, 0)),
            pl.BlockSpec((2, EMB), lambda i: (0, 0)),
        ],
        out_shape=[
            jax.ShapeDtypeStruct((n_rows, EMB), jnp.float32),
            jax.ShapeDtypeStruct((2, EMB), jnp.float32),
        ],
    )(agg, own, sl.reshape(1, EMB), w1, b1.reshape(1, 2 * EMB), w2,
      b2.reshape(1, EMB))
    return y, sums


def _css_body(y_ref, m_ref, o_ref):
    i = pl.program_id(0)
    d = y_ref[...] - m_ref[...]

    @pl.when(i == 0)
    def _():
        o_ref[...] = jnp.zeros_like(o_ref)

    o_ref[...] += jnp.sum(d * d, axis=0, keepdims=True)


def _colsumsq_centered(y, m, bm=2000):
    n = y.shape[0]
    nb = n // bm
    assert nb * bm == n
    return pl.pallas_call(
        _css_body,
        grid=(nb,),
        in_specs=[
            pl.BlockSpec((bm, EMB), lambda i: (i, 0)),
            pl.BlockSpec((1, EMB), lambda i: (0, 0)),
        ],
        out_specs=pl.BlockSpec((1, EMB), lambda i: (0, 0)),
        out_shape=jax.ShapeDtypeStruct((1, EMB), jnp.float32),
    )(y, m.reshape(1, EMB))


def _bn_apply_body(y_ref, sc_ref, sh_ref, o_ref, *, do_relu):
    out = y_ref[...] * sc_ref[...] + sh_ref[...]
    if do_relu:
        out = jnp.maximum(out, 0.0)
    o_ref[...] = out


def _bn_apply(y, scale, shift, do_relu, bm=2000):
    n = y.shape[0]
    nb = n // bm
    assert nb * bm == n
    return pl.pallas_call(
        functools.partial(_bn_apply_body, do_relu=do_relu),
        grid=(nb,),
        in_specs=[
            pl.BlockSpec((bm, EMB), lambda i: (i, 0)),
            pl.BlockSpec((1, EMB), lambda i: (0, 0)),
            pl.BlockSpec((1, EMB), lambda i: (0, 0)),
        ],
        out_specs=pl.BlockSpec((bm, EMB), lambda i: (i, 0)),
        out_shape=jax.ShapeDtypeStruct((n, EMB), jnp.float32),
    )(y, scale.reshape(1, EMB), shift.reshape(1, EMB))


def _bn_scale_shift(sums, y, n_rows, g, b):
    m = sums[0] / n_rows
    v = _colsumsq_centered(y, m)[0] / n_rows
    scale = g / jnp.sqrt(v + 1e-5)
    shift = b - m * scale
    return scale, shift


def _pad1(a, n, val):
    return jnp.pad(a, (0, n - a.shape[0]), constant_values=val)


def kernel(x, edge_index, edge_attr, lg_x, lg_edge_index, lg_edge_index_map,
           lg_edge_index_map2, params):
    i32 = jnp.int32
    x = x.astype(i32)
    edge_index = edge_index.astype(i32)
    edge_attr = edge_attr.astype(i32)
    lg_x = lg_x.astype(i32)
    lg_edge_index = lg_edge_index.astype(i32)
    lg_edge_index_map = lg_edge_index_map.astype(i32)
    lg_edge_index_map2 = lg_edge_index_map2.astype(i32)

    N = x.shape[0]
    M = lg_x.shape[0]

    NP, N_NCK, N_CH, N_W = 10240, 2, 5120, 320
    MP, M_NCK, M_CH, M_W = 80896, 8, 10112, 320
    EP = 163840
    ELGP = 327680
    conv_node = _make_conv(EP, NP, N_NCK, N_CH, N_W)
    conv_lg = _make_conv(ELGP, MP, M_NCK, M_CH, M_W)

    zrs = jnp.zeros((max(N_CH, M_CH), EMB), jnp.float32)
    arM = jnp.arange(M, dtype=i32)

    h0 = _embed(x[:, 0], x[:, 1], params['node_emb1'], params['node_emb2'])
    e0 = _embed(lg_x[:, 0], lg_x[:, 1], params['edge_emb1'],
                params['edge_emb2'])

    code_n0 = _pad1(edge_attr[:, 0] * 3 + edge_attr[:, 1], EP, 0)
    xg = x[lg_edge_index_map2]
    code_lg0 = _pad1(xg[:, 0] * 3 + xg[:, 1], ELGP, 0)
    ns1_0 = _pad1(edge_index[0], EP, 0)
    ndst_0 = _pad1(edge_index[1], EP, -1)
    ns1_12 = _pad1(jnp.concatenate([lg_edge_index_map[0], lg_edge_index_map[1]]), EP, 0)
    ndst_12 = _pad1(jnp.concatenate([lg_edge_index_map[1], lg_edge_index_map[0]]), EP, -1)
    ns2_12 = _pad1(jnp.concatenate([arM, arM]), EP, 0)
    lgs1 = _pad1(lg_edge_index[0], ELGP, 0)
    lgdst = _pad1(lg_edge_index[1], ELGP, -1)
    lgmap2 = _pad1(lg_edge_index_map2, ELGP, 0)

    h_cur = h0
    e_cur = e0
    h_list = [h0]
    e_list = [e0]

    for layer in range(NUM_LAYER):
        p = params['layers'][layer]
        q = params['lg_layers'][layer]
        do_relu = layer < NUM_LAYER - 1

        if layer == 0:
            t_n = (p['ee1'][:6, None, :] + p['ee2'][None, :3, :]).reshape(18, EMB)
            aggh = conv_node(h_cur, t_n, ns1_0, code_n0, ndst_0, zrs)
        else:
            aggh = conv_node(h_cur, e_cur, ns1_12, ns2_12, ndst_12, zrs)
        sl_n = p['ee1'][4] + p['ee2'][0]
        y_h, sums_h = _mlp(aggh, h_cur, sl_n, p['W1'], p['b1'], p['W2'],
                           p['b2'], N)
        sc_h, sh_h = _bn_scale_shift(sums_h, y_h, N, p['bn_g'], p['bn_b'])
        h_new = _bn_apply(y_h, sc_h, sh_h, do_relu)

        if layer == 0:
            t_l = (q['ee1'][:3, None, :] + q['ee2'][None, :3, :]).reshape(9, EMB)
            agge = conv_lg(e_cur, t_l, lgs1, code_lg0, lgdst, zrs)
        else:
            agge = conv_lg(e_cur, h_cur, lgs1, lgmap2, lgdst, zrs)
        sl_l = q['ee1'][119] + q['ee2'][0]
        y_e, sums_e = _mlp(agge, e_cur, sl_l, q['W1'], q['b1'], q['W2'],
                           q['b2'], M)
        sc_e, sh_e = _bn_scale_shift(sums_e, y_e, M, q['bn_g'], q['bn_b'])
        e_new = _bn_apply(y_e, sc_e, sh_e, do_relu)

        h_cur = h_new
        e_cur = e_new
        h_list.append(h_new)
        e_list.append(e_new)

    return (tuple(h_list), tuple(e_list))
